# baseline (speedup 1.0000x reference)
import functools

import jax
import jax.numpy as jnp
from jax import lax
from jax.experimental import pallas as pl
from jax.experimental.pallas import tpu as pltpu
from jax.experimental.pallas import tpu_sc as plsc

DE = 64
TOKEN_SIZE = 100000
BATCH = 4
SEQ = 2048

_info = plsc.get_sparse_core_info()
NC, NS = _info.num_cores, _info.num_subcores
NW = NC * NS
ROWS = BATCH * SEQ
RPW = ROWS // NW
ICH = 128
NCHUNK = RPW // ICH
SPB = SEQ // (NW // BATCH)


def _embed_kernel(idx_hbm, tok_hbm, pos_hbm, out_hbm, idx_v, rows_v, pos_v, sem):
    wid = lax.axis_index("s") * NC + lax.axis_index("c")
    base = wid * RPW
    s_off = (wid % (NW // BATCH)) * SPB

    pltpu.sync_copy(idx_hbm.at[wid], idx_v)
    gathers = [
        pltpu.async_copy(
            tok_hbm.at[idx_v.at[j]], rows_v.at[pl.ds(j * ICH, ICH)], sem
        )
        for j in range(NCHUNK)
    ]
    pltpu.sync_copy(pos_hbm.at[pl.ds(s_off, RPW)], pos_v)
    for g in gathers:
        g.wait()

    def add_row(r, carry):
        for j in range(DE // 16):
            sl = pl.ds(j * 16, 16)
            rows_v[r, sl] = rows_v[r, sl] + pos_v[r, sl]
        return carry

    lax.fori_loop(0, RPW, add_row, 0)
    pltpu.sync_copy(rows_v, out_hbm.at[pl.ds(base, RPW)])


@functools.partial(
    pl.kernel,
    mesh=plsc.VectorSubcoreMesh(core_axis_name="c", subcore_axis_name="s"),
    out_type=jax.ShapeDtypeStruct((ROWS, DE), jnp.float32),
    compiler_params=pltpu.CompilerParams(use_tc_tiling_on_sc=False),
    scratch_types=[
        pltpu.VMEM((NCHUNK, ICH), jnp.int32),
        pltpu.VMEM((RPW, DE), jnp.float32),
        pltpu.VMEM((RPW, DE), jnp.float32),
        pltpu.SemaphoreType.DMA,
    ],
)
def _embed(idx_hbm, tok_hbm, pos_hbm, out_hbm, idx_v, rows_v, pos_v, sem):
    _embed_kernel(idx_hbm, tok_hbm, pos_hbm, out_hbm, idx_v, rows_v, pos_v, sem)


def kernel(inputs, token_table, pos_table):
    idx = inputs.astype(jnp.int32).reshape(NW, NCHUNK, ICH)
    out = _embed(idx, token_table, pos_table)
    return out.reshape(BATCH, SEQ, DE)


# speedup vs baseline: 1.2507x; 1.2507x over previous
import functools

import jax
import jax.numpy as jnp
from jax import lax
from jax.experimental import pallas as pl
from jax.experimental.pallas import tpu as pltpu
from jax.experimental.pallas import tpu_sc as plsc

DE = 64
TOKEN_SIZE = 100000
BATCH = 4
SEQ = 2048

_info = plsc.get_sparse_core_info()
NC, NS = _info.num_cores, _info.num_subcores
NW = NC * NS
ROWS = BATCH * SEQ
RPW = ROWS // NW
CH = 32
NCHUNK = RPW // CH
SPB = SEQ // (NW // BATCH)
TBLK = 8


def _embed_body(hi_hbm, lo_hbm, tok_hbm, pos_hbm, out_hbm,
                hi_v, lo_v, blk0, blk1, rows_v, sem0, sem1):
    wid = lax.axis_index("s") * NC + lax.axis_index("c")
    base = wid * RPW
    s_off = (wid % (NW // BATCH)) * SPB

    pltpu.sync_copy(hi_hbm.at[wid], hi_v)
    pltpu.sync_copy(lo_hbm.at[wid], lo_v)

    blks = (blk0, blk1)
    sems = (sem0, sem1)

    def fire(c):
        blk, sem = blks[c % 2], sems[c % 2]

        def issue(g, carry):
            bs = hi_v[pl.ds(c * CH + g * 16, 16)]
            for l in range(16):
                pltpu.async_copy(
                    tok_hbm.at[pl.ds(bs[l], 1)],
                    blk.at[pl.ds(g * 16 + l, 1)], sem)
            return carry

        lax.fori_loop(0, CH // 16, issue, 0)

    def drain(c):
        blk, sem = blks[c % 2], sems[c % 2]
        pltpu.make_async_copy(tok_hbm.at[pl.ds(0, CH)], blk, sem).wait()

    fire(0)
    pltpu.sync_copy(pos_hbm.at[pl.ds(s_off, RPW)], rows_v)

    for c in range(NCHUNK):
        if c + 1 < NCHUNK:
            fire(c + 1)
        drain(c)
        blk = blks[c % 2]

        def select_add(g, carry, c=c, blk=blk):
            los = lo_v[pl.ds(c * CH + g * 16, 16)]
            for l in range(16):
                r = c * CH + g * 16 + l
                lo = los[l]
                for j in range(DE // 16):
                    sl = pl.ds(j * 16, 16)
                    rows_v[r, sl] = rows_v[r, sl] + blk[g * 16 + l, lo, sl]
            return carry

        lax.fori_loop(0, CH // 16, select_add, 0)

    pltpu.sync_copy(rows_v, out_hbm.at[pl.ds(base, RPW)])


@functools.partial(
    pl.kernel,
    mesh=plsc.VectorSubcoreMesh(core_axis_name="c", subcore_axis_name="s"),
    out_type=jax.ShapeDtypeStruct((ROWS, DE), jnp.float32),
    scratch_types=[
        pltpu.VMEM((RPW,), jnp.int32),
        pltpu.VMEM((RPW,), jnp.int32),
        pltpu.VMEM((CH, TBLK, DE), jnp.float32),
        pltpu.VMEM((CH, TBLK, DE), jnp.float32),
        pltpu.VMEM((RPW, DE), jnp.float32),
        pltpu.SemaphoreType.DMA,
        pltpu.SemaphoreType.DMA,
    ],
)
def _embed(hi_hbm, lo_hbm, tok_hbm, pos_hbm, out_hbm,
           hi_v, lo_v, blk0, blk1, rows_v, sem0, sem1):
    _embed_body(hi_hbm, lo_hbm, tok_hbm, pos_hbm, out_hbm,
                hi_v, lo_v, blk0, blk1, rows_v, sem0, sem1)


def kernel(inputs, token_table, pos_table):
    flat = inputs.astype(jnp.int32)
    hi = (flat // TBLK).reshape(NW, RPW)
    lo = (flat % TBLK).reshape(NW, RPW)
    tok3 = token_table.reshape(TOKEN_SIZE // TBLK, TBLK, DE)
    out = _embed(hi, lo, tok3, pos_table)
    return out.reshape(BATCH, SEQ, DE)


# speedup vs baseline: 2.0554x; 1.6434x over previous
import functools

import jax
import jax.numpy as jnp
from jax import lax
from jax.experimental import pallas as pl
from jax.experimental.pallas import tpu as pltpu
from jax.experimental.pallas import tpu_sc as plsc

DE = 64
TOKEN_SIZE = 100000
BATCH = 4
SEQ = 2048

_info = plsc.get_sparse_core_info()
NC, NS = _info.num_cores, _info.num_subcores
NW = NC * NS
NPASS = DE // NW
GRP = SEQ // 16


def _embed_body(idx_hbm, tok_hbm, pos_hbm, out_hbm, idx_v, row_v, pos_v, out_v):
    wid = lax.axis_index("s") * NC + lax.axis_index("c")

    pltpu.sync_copy(idx_hbm, idx_v)

    for p in range(NPASS):
        d = p * NW + wid
        pltpu.sync_copy(tok_hbm.at[d], row_v)
        pltpu.sync_copy(pos_hbm.at[d], pos_v)

        def gather_add(g, carry):
            sl = pl.ds(g * 16, 16)
            pv = pos_v[sl]
            for b in range(BATCH):
                ids = idx_v[pl.ds(b * SEQ + g * 16, 16)]
                out_v[b, sl] = plsc.load_gather(row_v, [ids]) + pv
            return carry

        lax.fori_loop(0, GRP, gather_add, 0)

        for b in range(BATCH):
            pltpu.sync_copy(out_v.at[b], out_hbm.at[b, d])


@functools.partial(
    pl.kernel,
    mesh=plsc.VectorSubcoreMesh(core_axis_name="c", subcore_axis_name="s"),
    out_type=jax.ShapeDtypeStruct((BATCH, DE, SEQ), jnp.float32),
    compiler_params=pltpu.CompilerParams(needs_layout_passes=False),
    scratch_types=[
        pltpu.VMEM((BATCH * SEQ,), jnp.int32),
        pltpu.VMEM((TOKEN_SIZE,), jnp.float32),
        pltpu.VMEM((SEQ,), jnp.float32),
        pltpu.VMEM((BATCH, SEQ), jnp.float32),
    ],
)
def _embed(idx_hbm, tok_hbm, pos_hbm, out_hbm, idx_v, row_v, pos_v, out_v):
    _embed_body(idx_hbm, tok_hbm, pos_hbm, out_hbm, idx_v, row_v, pos_v, out_v)


def kernel(inputs, token_table, pos_table):
    idx = inputs.astype(jnp.int32).reshape(BATCH * SEQ)
    out = _embed(idx, token_table.T, pos_table.T)
    return jnp.transpose(out, (0, 2, 1))
